# baseline (speedup 1.0000x reference)
import functools

import numpy as np
import jax
import jax.numpy as jnp
from jax import lax
from jax.experimental import pallas as pl
from jax.experimental.pallas import tpu as pltpu

N_IN = 64
N_OUT = 16
SEQ = 4096
DM = 128

_GUMBEL = np.asarray(
    jax.random.gumbel(jax.random.key(42), (N_IN,), jnp.float32))


def _tc_body(w_smem, states_hbm, out_hbm, scores_smem, sem):
  for i in range(N_IN):
    scores_smem[i] = w_smem[i] + float(_GUMBEL[i])

  def pick(k, _):
    def scan_i(i, carry):
      best, bi = carry
      v = scores_smem[i]
      take = v > best
      return (jnp.where(take, v, best), jnp.where(take, i, bi))
    best, bi = lax.fori_loop(0, N_IN, scan_i, (jnp.float32(-jnp.inf),
                                               jnp.int32(0)))
    scores_smem[bi] = -jnp.inf
    pltpu.make_async_copy(states_hbm.at[bi], out_hbm.at[k], sem).start()
    return 0

  lax.fori_loop(0, N_OUT, pick, 0)
  for _ in range(N_OUT):
    pltpu.make_async_copy(states_hbm.at[0], out_hbm.at[0], sem).wait()


@jax.jit
def _merge_filter_tc(states, w_merge):
  return pl.pallas_call(
      _tc_body,
      out_shape=jax.ShapeDtypeStruct((N_OUT, SEQ, DM), jnp.float32),
      in_specs=[
          pl.BlockSpec(memory_space=pltpu.SMEM),
          pl.BlockSpec(memory_space=pl.ANY),
      ],
      out_specs=pl.BlockSpec(memory_space=pl.ANY),
      scratch_shapes=[
          pltpu.SMEM((N_IN,), jnp.float32),
          pltpu.SemaphoreType.DMA,
      ],
  )(w_merge, states)


def kernel(states, w_merge):
  return _merge_filter_tc(states, w_merge)


# speedup vs baseline: 22.0208x; 22.0208x over previous
import functools

import numpy as np
import jax
import jax.numpy as jnp
from jax import lax
from jax.experimental import pallas as pl
from jax.experimental.pallas import tpu as pltpu
from jax.experimental.pallas import tpu_sc as plsc

N_IN = 64
N_OUT = 16
SEQ = 4096
DM = 128

L = 16
NC = 2
NS = 16
NW = NC * NS

SEQ_PER_TILE = SEQ // 2
BATCH = 128
NBATCH = SEQ_PER_TILE // BATCH
NBUF = 6


def _top16(w_v, g_v):
  def merge(ka, va, kb, vb):
    kb_r = lax.rev(kb, (0,))
    vb_r = lax.rev(vb, (0,))
    take_a = ka >= kb_r
    km = jnp.where(take_a, ka, kb_r)
    vm = jnp.where(take_a, va, vb_r)
    return plsc.sort_key_val(km, vm, descending=True)

  ks, vs = [], []
  for i in range(N_IN // L):
    s = w_v[pl.ds(i * L, L)] + g_v[pl.ds(i * L, L)]
    idx = lax.iota(jnp.int32, L) + i * L
    k, v = plsc.sort_key_val(s, idx, descending=True)
    ks.append(k)
    vs.append(v)
  k01, v01 = merge(ks[0], vs[0], ks[1], vs[1])
  k23, v23 = merge(ks[2], vs[2], ks[3], vs[3])
  _, top = merge(k01, v01, k23, v23)
  return top


def _body(states_hbm, w_hbm, g_hbm, out_hbm, w_v, g_v, buf_v, gsem, ssem):
  wid = lax.axis_index("s") * NC + lax.axis_index("c")

  pltpu.sync_copy(w_hbm, w_v)
  pltpu.sync_copy(g_hbm, g_v)
  topidx = _top16(w_v, g_v)

  p = lax.div(wid, 2)
  half = lax.rem(wid, 2)
  lanes = lax.iota(jnp.int32, L)
  row = lax.reduce_max(jnp.where(lanes == p, topidx, 0), axes=(0,))
  seq0 = half * SEQ_PER_TILE
  sid = lax.axis_index("s")
  buf_v = buf_v.at[sid]

  def start_gather(b):
    return pltpu.async_copy(
        states_hbm.at[row, pl.ds(seq0 + b * BATCH, BATCH)],
        buf_v.at[b % NBUF], gsem)

  gathers = [None] * NBATCH
  scatters = [None] * NBATCH
  for b in range(NBUF - 1):
    gathers[b] = start_gather(b)
  for b in range(NBATCH):
    if b + NBUF - 1 < NBATCH:
      if b >= 1:
        scatters[b - 1].wait()
      gathers[b + NBUF - 1] = start_gather(b + NBUF - 1)
    gathers[b].wait()
    scatters[b] = pltpu.async_copy(
        buf_v.at[b % NBUF],
        out_hbm.at[p, pl.ds(seq0 + b * BATCH, BATCH)], ssem)
  for b in range(NBATCH - NBUF, NBATCH):
    if b >= 0:
      scatters[b].wait()


@jax.jit
def _merge_filter(states, w_merge, gumbel):
  mesh = plsc.VectorSubcoreMesh(core_axis_name="c", subcore_axis_name="s")
  run = functools.partial(
      pl.kernel,
      out_type=jax.ShapeDtypeStruct((N_OUT, SEQ, DM), jnp.float32),
      mesh=mesh,
      scratch_types=[
          pltpu.VMEM((N_IN,), jnp.float32),
          pltpu.VMEM((N_IN,), jnp.float32),
          pltpu.VMEM_SHARED((NS, NBUF, BATCH, DM), jnp.float32),
          pltpu.SemaphoreType.DMA,
          pltpu.SemaphoreType.DMA,
      ],
      compiler_params=pltpu.CompilerParams(needs_layout_passes=False),
  )(_body)
  return run(states, w_merge, gumbel)


_GUMBEL = np.asarray(
    jax.random.gumbel(jax.random.key(42), (N_IN,), jnp.float32))


def kernel(states, w_merge):
  return _merge_filter(states, w_merge, jnp.asarray(_GUMBEL))


# speedup vs baseline: 23.3456x; 1.0602x over previous
import functools

import numpy as np
import jax
import jax.numpy as jnp
from jax import lax
from jax.experimental import pallas as pl
from jax.experimental.pallas import tpu as pltpu
from jax.experimental.pallas import tpu_sc as plsc

N_IN = 64
N_OUT = 16
SEQ = 4096
DM = 128

L = 16
NC = 2
NS = 16
NW = NC * NS

SEQ_PER_TILE = SEQ // 2
BATCH = 256
NBATCH = SEQ_PER_TILE // BATCH
NBUF = 3


def _top16(w_v, g_v):
  def merge(ka, va, kb, vb):
    kb_r = lax.rev(kb, (0,))
    vb_r = lax.rev(vb, (0,))
    take_a = ka >= kb_r
    km = jnp.where(take_a, ka, kb_r)
    vm = jnp.where(take_a, va, vb_r)
    return plsc.sort_key_val(km, vm, descending=True)

  ks, vs = [], []
  for i in range(N_IN // L):
    s = w_v[pl.ds(i * L, L)] + g_v[pl.ds(i * L, L)]
    idx = lax.iota(jnp.int32, L) + i * L
    k, v = plsc.sort_key_val(s, idx, descending=True)
    ks.append(k)
    vs.append(v)
  k01, v01 = merge(ks[0], vs[0], ks[1], vs[1])
  k23, v23 = merge(ks[2], vs[2], ks[3], vs[3])
  _, top = merge(k01, v01, k23, v23)
  return top


def _body(states_hbm, w_hbm, g_hbm, out_hbm, w_v, g_v, buf_v, gsem, ssem):
  wid = lax.axis_index("s") * NC + lax.axis_index("c")

  cw = pltpu.async_copy(w_hbm, w_v, gsem)
  cg = pltpu.async_copy(g_hbm, g_v, ssem)
  cw.wait()
  cg.wait()
  topidx = _top16(w_v, g_v)

  p = lax.div(wid, 2)
  half = lax.rem(wid, 2)
  lanes = lax.iota(jnp.int32, L)
  row = lax.reduce_max(jnp.where(lanes == p, topidx, 0), axes=(0,))
  seq0 = half * SEQ_PER_TILE

  def start_gather(b):
    return pltpu.async_copy(
        states_hbm.at[row, pl.ds(seq0 + b * BATCH, BATCH)],
        buf_v.at[b % NBUF], gsem)

  gathers = [None] * NBATCH
  scatters = [None] * NBATCH
  for b in range(NBUF - 1):
    gathers[b] = start_gather(b)
  for b in range(NBATCH):
    if b + NBUF - 1 < NBATCH:
      if b >= 1:
        scatters[b - 1].wait()
      gathers[b + NBUF - 1] = start_gather(b + NBUF - 1)
    gathers[b].wait()
    scatters[b] = pltpu.async_copy(
        buf_v.at[b % NBUF],
        out_hbm.at[p, pl.ds(seq0 + b * BATCH, BATCH)], ssem)
  for b in range(NBATCH - NBUF, NBATCH):
    if b >= 0:
      scatters[b].wait()


@jax.jit
def _merge_filter(states, w_merge, gumbel):
  mesh = plsc.VectorSubcoreMesh(core_axis_name="c", subcore_axis_name="s")
  run = functools.partial(
      pl.kernel,
      out_type=jax.ShapeDtypeStruct((N_OUT, SEQ, DM), jnp.float32),
      mesh=mesh,
      scratch_types=[
          pltpu.VMEM((N_IN,), jnp.float32),
          pltpu.VMEM((N_IN,), jnp.float32),
          pltpu.VMEM((NBUF, BATCH, DM), jnp.float32),
          pltpu.SemaphoreType.DMA,
          pltpu.SemaphoreType.DMA,
      ],
      compiler_params=pltpu.CompilerParams(needs_layout_passes=False),
  )(_body)
  return run(states, w_merge, gumbel)


_GUMBEL = np.asarray(
    jax.random.gumbel(jax.random.key(42), (N_IN,), jnp.float32))


def kernel(states, w_merge):
  return _merge_filter(states, w_merge, jnp.asarray(_GUMBEL))
